# Initial kernel scaffold; baseline (speedup 1.0000x reference)
import jax
import jax.numpy as jnp
from jax.experimental import pallas as pl


def kernel(hidden_states, W_proj, b_proj, codebook):
    raise NotImplementedError("write your pallas kernel here")


# speedup vs baseline: 3.7944x; 3.7944x over previous
import functools

import jax
import jax.numpy as jnp
from jax import lax
from jax.experimental import pallas as pl
from jax.experimental.pallas import tpu as pltpu
from jax.experimental.pallas import tpu_sc as plsc


def _tc_proj_argmax(x2d, W, b2d, G, V, rows_per_step):
    BT, DIM = x2d.shape
    GV = G * V
    nsteps = BT // rows_per_step

    def body(x_ref, w_ref, b_ref, idx_ref, cnt_ref):
        i = pl.program_id(0)
        h = jnp.dot(x_ref[...], w_ref[...], preferred_element_type=jnp.float32)
        h = h + b_ref[...]

        @pl.when(i == 0)
        def _():
            cnt_ref[...] = jnp.zeros_like(cnt_ref)

        R = x_ref.shape[0]
        iota = lax.broadcasted_iota(jnp.int32, (R, V), 1)
        for g in range(G):
            hg = h[:, g * V:(g + 1) * V]
            mg = jnp.max(hg, axis=1, keepdims=True)
            ig = jnp.min(jnp.where(hg == mg, iota, V), axis=1)
            idx_ref[g, :] = ig + g * V
            onehot = (iota == ig[:, None]).astype(jnp.float32)
            cnt_ref[g, :] += jnp.sum(onehot, axis=0)

        @pl.when(i == nsteps - 1)
        def _():
            cnt_ref[...] = cnt_ref[...] * (1.0 / BT)

    return pl.pallas_call(
        body,
        grid=(nsteps,),
        in_specs=[
            pl.BlockSpec((rows_per_step, DIM), lambda i: (i, 0)),
            pl.BlockSpec((DIM, GV), lambda i: (0, 0)),
            pl.BlockSpec((1, GV), lambda i: (0, 0)),
        ],
        out_specs=[
            pl.BlockSpec((G, rows_per_step), lambda i: (0, i)),
            pl.BlockSpec((G, V), lambda i: (0, 0)),
        ],
        out_shape=[
            jax.ShapeDtypeStruct((G, BT), jnp.int32),
            jax.ShapeDtypeStruct((G, V), jnp.float32),
        ],
    )(x2d, W, b2d)


def _sc_gather(table, flat_idx, D):
    info = plsc.get_sparse_core_info()
    NC, NS = info.num_cores, info.num_subcores
    NW = NC * NS
    N = flat_idx.shape[0]
    assert N % (8 * NW) == 0
    b_per_w = N // NW
    chunk = 256
    nchunks = b_per_w // chunk
    mesh = plsc.VectorSubcoreMesh(core_axis_name="c", subcore_axis_name="s")

    @functools.partial(
        pl.kernel,
        mesh=mesh,
        out_type=jax.ShapeDtypeStruct((N, D), jnp.float32),
        scratch_types=[
            pltpu.VMEM((b_per_w,), jnp.int32),
            pltpu.VMEM((chunk, D), jnp.float32),
            pltpu.SemaphoreType.DMA,
        ],
    )
    def k(table_hbm, idx_hbm, out_hbm, idx_v, rows_v, sem):
        wid = lax.axis_index("s") * NC + lax.axis_index("c")
        base = wid * b_per_w
        pltpu.sync_copy(idx_hbm.at[pl.ds(base, b_per_w)], idx_v)
        for c in range(nchunks):
            pltpu.async_copy(
                table_hbm.at[idx_v.at[pl.ds(c * chunk, chunk)]], rows_v, sem
            ).wait()
            pltpu.sync_copy(rows_v, out_hbm.at[pl.ds(base + c * chunk, chunk)])

    return k(table, flat_idx)


def kernel(hidden_states, W_proj, b_proj, codebook):
    B, T, DIM = hidden_states.shape
    GV, D = codebook.shape
    V = 1024
    G = GV // V
    BT = B * T

    x2d = hidden_states.reshape(BT, DIM)
    b2d = b_proj.reshape(1, GV)
    idx2, perplexity = _tc_proj_argmax(x2d, W_proj, b2d, G, V, rows_per_step=512)
    flat_idx = idx2.T.reshape(BT * G)
    gathered = _sc_gather(codebook, flat_idx, D)
    code_vectors = gathered.reshape(B, T, G * D)
    return (code_vectors, perplexity)


# speedup vs baseline: 5.0530x; 1.3317x over previous
import functools

import jax
import jax.numpy as jnp
from jax import lax
from jax.experimental import pallas as pl
from jax.experimental.pallas import tpu as pltpu
from jax.experimental.pallas import tpu_sc as plsc


def _tc_proj_argmax(x2d, W, b2d, G, V, rows_per_step):
    BT, DIM = x2d.shape
    GV = G * V
    nsteps = BT // rows_per_step

    R = rows_per_step

    def body(x_ref, w_ref, b_ref, idx0_ref, idx1_ref, cnt_ref, h_s):
        idx_refs = (idx0_ref, idx1_ref)
        i = pl.program_id(0)
        j = lax.rem(i, 2)
        hp = h_s[1 - j]

        @pl.when(i == 1)
        def _():
            cnt_ref[...] = jnp.zeros_like(cnt_ref)

        iota_v = lax.broadcasted_iota(jnp.int32, (R, V), 1)
        for g, idx_ref in enumerate(idx_refs):
            hg = hp[:, g * V:(g + 1) * V]
            ig = jnp.argmax(hg, axis=1).astype(jnp.int32)
            idx_ref[...] = ig + g * V
            onehot = (iota_v == ig[:, None]).astype(jnp.float32)
            cnt_ref[g, :] += jnp.sum(onehot, axis=0)

        h = jnp.dot(x_ref[...], w_ref[...], preferred_element_type=jnp.float32)
        h_s[j] = h + b_ref[...]

        @pl.when(i == nsteps)
        def _():
            cnt_ref[...] = cnt_ref[...] * (1.0 / BT)

    last = nsteps - 1
    return pl.pallas_call(
        body,
        grid=(nsteps + 1,),
        in_specs=[
            pl.BlockSpec((R, DIM), lambda i: (jnp.minimum(i, last), 0)),
            pl.BlockSpec((DIM, GV), lambda i: (0, 0)),
            pl.BlockSpec((1, GV), lambda i: (0, 0)),
        ],
        out_specs=[
            pl.BlockSpec((R,), lambda i: (jnp.maximum(i - 1, 0),)),
            pl.BlockSpec((R,), lambda i: (jnp.maximum(i - 1, 0),)),
            pl.BlockSpec((G, V), lambda i: (0, 0)),
        ],
        out_shape=[
            jax.ShapeDtypeStruct((BT,), jnp.int32),
            jax.ShapeDtypeStruct((BT,), jnp.int32),
            jax.ShapeDtypeStruct((G, V), jnp.float32),
        ],
        scratch_shapes=[pltpu.VMEM((2, R, GV), jnp.float32)],
    )(x2d, W, b2d)


def _sc_gather(table, idx0, idx1, D):
    info = plsc.get_sparse_core_info()
    NC, NS = info.num_cores, info.num_subcores
    NW = NC * NS
    BT = idx0.shape[0]
    G = 2
    assert BT % (8 * NW) == 0
    t_per_w = BT // NW
    chunk = 64
    NB = 4
    nchunks = t_per_w // chunk
    items = [(c, g) for c in range(nchunks) for g in range(G)]
    mesh = plsc.VectorSubcoreMesh(core_axis_name="c", subcore_axis_name="s")

    @functools.partial(
        pl.kernel,
        mesh=mesh,
        out_type=jax.ShapeDtypeStruct((BT, G * D), jnp.float32),
        scratch_types=[
            pltpu.VMEM((t_per_w,), jnp.int32),
            pltpu.VMEM((t_per_w,), jnp.int32),
        ]
        + [pltpu.VMEM((chunk, D), jnp.float32) for _ in range(NB)]
        + [pltpu.SemaphoreType.DMA for _ in range(2 * NB)],
    )
    def k(table_hbm, idx0_hbm, idx1_hbm, out_hbm, v0, v1, *bufs_and_sems):
        bufs = bufs_and_sems[:NB]
        gsem = bufs_and_sems[NB:2 * NB]
        wsem = bufs_and_sems[2 * NB:3 * NB]
        wid = lax.axis_index("s") * NC + lax.axis_index("c")
        tb = wid * t_per_w
        pltpu.sync_copy(idx0_hbm.at[pl.ds(tb, t_per_w)], v0)
        pltpu.sync_copy(idx1_hbm.at[pl.ds(tb, t_per_w)], v1)
        idx_refs = (v0, v1)

        def gather(item, s):
            c, g = item
            return pltpu.async_copy(
                table_hbm.at[idx_refs[g].at[pl.ds(c * chunk, chunk)]],
                bufs[s], gsem[s])

        def writeback(item, s):
            c, g = item
            return pltpu.async_copy(
                bufs[s],
                out_hbm.at[pl.ds(tb + c * chunk, chunk), pl.ds(g * D, D)],
                wsem[s])

        gcp = [None] * NB
        wcp = [None] * NB
        for it in range(NB):
            gcp[it] = gather(items[it], it)
        for it in range(len(items)):
            s = it % NB
            gcp[s].wait()
            wcp[s] = writeback(items[it], s)
            n = it + NB
            if n < len(items):
                wcp[s].wait()
                gcp[s] = gather(items[n], s)
        for s in range(NB):
            if wcp[s] is not None:
                wcp[s].wait()

    return k(table, idx0, idx1)


def kernel(hidden_states, W_proj, b_proj, codebook):
    B, T, DIM = hidden_states.shape
    GV, D = codebook.shape
    V = 1024
    G = GV // V
    BT = B * T

    x2d = hidden_states.reshape(BT, DIM)
    b2d = b_proj.reshape(1, GV)
    idx0, idx1, perplexity = _tc_proj_argmax(
        x2d, W_proj, b2d, G, V, rows_per_step=512)
    gathered = _sc_gather(codebook, idx0, idx1, D)
    code_vectors = gathered.reshape(B, T, G * D)
    return (code_vectors, perplexity)
